# baseline (speedup 1.0000x reference)
import functools

import jax
import jax.numpy as jnp
from jax import lax
from jax.experimental import pallas as pl
from jax.experimental.pallas import tpu as pltpu
from jax.experimental.pallas import tpu_sc as plsc

N_NODES = 10000
D_EDGE = 16
NC = 2
NS = 16
LANES = 16

N_EDGES = 320000
E_HALF = N_EDGES // 2
ROWS_PER_STREAM = 128
N_CHUNKS_H = E_HALF // ROWS_PER_STREAM
STREAMS_PER_GROUP = 10
GROUP_ROWS = ROWS_PER_STREAM * STREAMS_PER_GROUP
N_GROUPS_H = N_CHUNKS_H // STREAMS_PER_GROUP
FULL_TILES = 29
DUMP_TILES = 10
DUMP_ROWS = N_NODES // DUMP_TILES


def _sc_body(edge0, ea_hbm, ei_hbm, out_hbm, idxf_v, idx_v, ea_v, zbuf,
             shared, sem):
    c = lax.axis_index("c")
    s = lax.axis_index("s")

    zstart = 624 * s
    zsize = jnp.where(s < NS - 1, 624, 640)

    def _zero(i, _):
        zbuf[i, :] = jnp.zeros((LANES,), jnp.float32)
        return 0

    lax.fori_loop(0, zsize, _zero, 0)
    pltpu.sync_copy(zbuf.at[pl.ds(0, 624)], shared.at[pl.ds(zstart, 624)])

    @pl.when(s == NS - 1)
    def _():
        pltpu.sync_copy(zbuf.at[pl.ds(624, 16)],
                        shared.at[pl.ds(624 * NS, 16)])

    plsc.subcore_barrier()

    t = c * NS + s
    gstart = jnp.where(t < FULL_TILES, 4 * t, 4 * FULL_TILES + 3 * (t - FULL_TILES))
    ngroups = jnp.where(t < FULL_TILES, 4, 3)

    def _group(g, _):
        ck = (gstart + g) * STREAMS_PER_GROUP
        pltpu.sync_copy(
            ei_hbm.at[pl.ds(edge0 + ck * ROWS_PER_STREAM, GROUP_ROWS)],
            idxf_v)
        pltpu.sync_copy(
            ea_hbm.at[pl.ds(ck * ROWS_PER_STREAM, GROUP_ROWS), pl.ds(0, D_EDGE)],
            ea_v)
        for j in range(STREAMS_PER_GROUP):
            for k in range(ROWS_PER_STREAM // LANES):
                idx_v[j, pl.ds(k * LANES, LANES)] = (
                    idxf_v[pl.ds(j * ROWS_PER_STREAM + k * LANES, LANES)])
        handles = [
            pltpu.async_copy(
                ea_v.at[pl.ds(j * ROWS_PER_STREAM, ROWS_PER_STREAM)],
                shared.at[idx_v.at[j]],
                sem,
                add=True,
            )
            for j in range(STREAMS_PER_GROUP)
        ]
        for h in handles:
            h.wait()
        return 0

    lax.fori_loop(0, ngroups, _group, 0)
    plsc.subcore_barrier()

    pltpu.sync_copy(
        shared.at[pl.ds(zstart, 624)],
        out_hbm.at[c, pl.ds(zstart, 624)],
    )

    @pl.when(s == NS - 1)
    def _():
        pltpu.sync_copy(
            shared.at[pl.ds(624 * NS, 16)],
            out_hbm.at[c, pl.ds(624 * NS, 16)],
        )


@functools.partial(jax.jit, static_argnames=("edge0",))
def _sc_scatter_add(ea_pad, col2d, edge0):
    mesh = plsc.VectorSubcoreMesh(core_axis_name="c", subcore_axis_name="s")
    return pl.kernel(
        functools.partial(_sc_body, edge0),
        out_type=jax.ShapeDtypeStruct((NC, N_NODES, D_EDGE), jnp.float32),
        mesh=mesh,
        scratch_types=[
            pltpu.VMEM((GROUP_ROWS,), jnp.int32),
            pltpu.VMEM((STREAMS_PER_GROUP, ROWS_PER_STREAM), jnp.int32),
            pltpu.VMEM((GROUP_ROWS, D_EDGE), jnp.float32),
            pltpu.VMEM((640, D_EDGE), jnp.float32),
            pltpu.VMEM_SHARED((N_NODES, D_EDGE), jnp.float32),
            pltpu.SemaphoreType.DMA,
        ],
        compiler_params=pltpu.CompilerParams(use_tc_tiling_on_sc=False),
    )(ea_pad, col2d)


TR_BLK = 6400


def _tr_body(ea_t_ref, out_ref):
    t = ea_t_ref[...]
    eye = jnp.eye(D_EDGE, dtype=jnp.float32)
    out_ref[:, pl.ds(0, D_EDGE)] = lax.dot_general(
        t, eye, (((0,), (0,)), ((), ())),
        preferred_element_type=jnp.float32)


@functools.partial(jax.jit, static_argnames=("half",))
def _tc_transpose(ea_t, half):
    nblk = E_HALF // TR_BLK
    blk0 = half * nblk
    return pl.pallas_call(
        _tr_body,
        grid=(nblk,),
        in_specs=[pl.BlockSpec((D_EDGE, TR_BLK), lambda i: (0, i + blk0))],
        out_specs=pl.BlockSpec((TR_BLK, 128), lambda i: (i, 0)),
        out_shape=jax.ShapeDtypeStruct((E_HALF, 128), jnp.float32),
    )(ea_t)


def _col_body(ei_ref, col_ref):
    col_ref[...] = ei_ref[1, :]


@jax.jit
def _tc_col(ei):
    return pl.pallas_call(
        _col_body,
        in_specs=[pl.BlockSpec((2, N_EDGES), lambda: (0, 0))],
        out_specs=pl.BlockSpec((N_EDGES,), lambda: (0,)),
        out_shape=jax.ShapeDtypeStruct((N_EDGES,), jnp.int32),
    )(ei)


def _tc_body(x_ref, pa_ref, pb_ref, batch_ref, u_ref, w1x_ref, w1e_ref,
             w1u_ref, b1_ref, w2_ref, b2_ref, out_ref):
    B = x_ref.shape[0]
    agg = (pa_ref[0] + pa_ref[1]) + (pb_ref[0] + pb_ref[1])
    g = jnp.dot(u_ref[...], w1u_ref[...], preferred_element_type=jnp.float32)
    onehot_t = (batch_ref[0] == lax.broadcasted_iota(jnp.int32, (16, B), 0)
                ).astype(jnp.float32)
    ug = lax.dot_general(onehot_t, g, (((0,), (0,)), ((), ())),
                         preferred_element_type=jnp.float32)
    pre = (jnp.dot(x_ref[...], w1x_ref[...], preferred_element_type=jnp.float32)
           + jnp.dot(agg, w1e_ref[...], preferred_element_type=jnp.float32)
           + ug
           + b1_ref[...])
    h = pre * jax.nn.sigmoid(pre)
    pre2 = jnp.dot(h, w2_ref[...], preferred_element_type=jnp.float32) + b2_ref[...]
    out_ref[...] = pre2 * jax.nn.sigmoid(pre2)


@functools.partial(jax.jit, static_argnames=("block",))
def _tc_mlp(x, parts_a, parts_b, batch2d, u, w1x, w1e, w1u, b1, w2, b2,
            block=1000):
    nblk = N_NODES // block
    k = w1u.shape[1]
    return pl.pallas_call(
        _tc_body,
        grid=(nblk,),
        in_specs=[
            pl.BlockSpec((block, x.shape[1]), lambda i: (i, 0)),
            pl.BlockSpec((NC, block, D_EDGE), lambda i: (0, i, 0)),
            pl.BlockSpec((NC, block, D_EDGE), lambda i: (0, i, 0)),
            pl.BlockSpec((1, 1, block), lambda i: (i, 0, 0)),
            pl.BlockSpec(u.shape, lambda i: (0, 0)),
            pl.BlockSpec(w1x.shape, lambda i: (0, 0)),
            pl.BlockSpec(w1e.shape, lambda i: (0, 0)),
            pl.BlockSpec(w1u.shape, lambda i: (0, 0)),
            pl.BlockSpec(b1.shape, lambda i: (0, 0)),
            pl.BlockSpec(w2.shape, lambda i: (0, 0)),
            pl.BlockSpec(b2.shape, lambda i: (0, 0)),
        ],
        out_specs=pl.BlockSpec((block, k), lambda i: (i, 0)),
        out_shape=jax.ShapeDtypeStruct((N_NODES, k), jnp.float32),
    )(x, parts_a, parts_b, batch2d, u, w1x, w1e, w1u, b1, w2, b2)


def kernel(x, edge_index, edge_attr, u, batch, W1, b1, W2, b2):
    col = _tc_col(edge_index.astype(jnp.int32))
    ea_t = edge_attr.T
    ea_a = _tc_transpose(ea_t, 0)
    parts_a = _sc_scatter_add(ea_a, col, 0)
    ea_b = _tc_transpose(ea_t, 1)
    parts_b = _sc_scatter_add(ea_b, col, E_HALF)

    d_feat = x.shape[1]
    w1x = W1[:d_feat]
    w1e = W1[d_feat:d_feat + D_EDGE]
    w1u = W1[d_feat + D_EDGE:]
    batch2d = batch.astype(jnp.int32).reshape(-1, 1, 1000)
    return _tc_mlp(x, parts_a, parts_b, batch2d, u, w1x, w1e, w1u,
                   b1.reshape(1, -1), W2, b2.reshape(1, -1))


# speedup vs baseline: 1.0913x; 1.0913x over previous
import functools

import jax
import jax.numpy as jnp
from jax import lax
from jax.experimental import pallas as pl
from jax.experimental.pallas import tpu as pltpu
from jax.experimental.pallas import tpu_sc as plsc

N_NODES = 10000
D_EDGE = 16
NC = 2
NS = 16
LANES = 16

N_EDGES = 320000
E_HALF = N_EDGES // 2
ROWS_PER_STREAM = 128
N_CHUNKS_H = E_HALF // ROWS_PER_STREAM
STREAMS_PER_GROUP = 10
GROUP_ROWS = ROWS_PER_STREAM * STREAMS_PER_GROUP
N_GROUPS_H = N_CHUNKS_H // STREAMS_PER_GROUP
FULL_TILES = 29
DUMP_TILES = 10
DUMP_ROWS = N_NODES // DUMP_TILES


def _sc_body(edge0, ea_hbm, ei_hbm, out_hbm, idxf_v, idx_v, ea_v, zbuf,
             shared, sem):
    c = lax.axis_index("c")
    s = lax.axis_index("s")

    zstart = 624 * s
    zsize = jnp.where(s < NS - 1, 624, 640)

    def _zero(i, _):
        zbuf[i, :] = jnp.zeros((LANES,), jnp.float32)
        return 0

    lax.fori_loop(0, zsize, _zero, 0)
    pltpu.sync_copy(zbuf.at[pl.ds(0, 624)], shared.at[pl.ds(zstart, 624)])

    @pl.when(s == NS - 1)
    def _():
        pltpu.sync_copy(zbuf.at[pl.ds(624, 16)],
                        shared.at[pl.ds(624 * NS, 16)])

    plsc.subcore_barrier()

    t = c * NS + s
    gstart = jnp.where(t < FULL_TILES, 4 * t, 4 * FULL_TILES + 3 * (t - FULL_TILES))
    ngroups = jnp.where(t < FULL_TILES, 4, 3)

    def _group(g, _):
        ck = (gstart + g) * STREAMS_PER_GROUP
        pltpu.sync_copy(
            ei_hbm.at[pl.ds(edge0 + ck * ROWS_PER_STREAM, GROUP_ROWS)],
            idxf_v)
        pltpu.sync_copy(
            ea_hbm.at[pl.ds(ck * ROWS_PER_STREAM, GROUP_ROWS), pl.ds(0, D_EDGE)],
            ea_v)
        for j in range(STREAMS_PER_GROUP):
            for k in range(ROWS_PER_STREAM // LANES):
                idx_v[j, pl.ds(k * LANES, LANES)] = (
                    idxf_v[pl.ds(j * ROWS_PER_STREAM + k * LANES, LANES)])
        handles = [
            pltpu.async_copy(
                ea_v.at[pl.ds(j * ROWS_PER_STREAM, ROWS_PER_STREAM)],
                shared.at[idx_v.at[j]],
                sem,
                add=True,
            )
            for j in range(STREAMS_PER_GROUP)
        ]
        for h in handles:
            h.wait()
        return 0

    lax.fori_loop(0, ngroups, _group, 0)
    plsc.subcore_barrier()

    pltpu.sync_copy(
        shared.at[pl.ds(zstart, 624)],
        out_hbm.at[c, pl.ds(zstart, 624)],
    )

    @pl.when(s == NS - 1)
    def _():
        pltpu.sync_copy(
            shared.at[pl.ds(624 * NS, 16)],
            out_hbm.at[c, pl.ds(624 * NS, 16)],
        )


@functools.partial(jax.jit, static_argnames=("edge0",))
def _sc_scatter_add(ea_pad, col2d, edge0):
    mesh = plsc.VectorSubcoreMesh(core_axis_name="c", subcore_axis_name="s")
    return pl.kernel(
        functools.partial(_sc_body, edge0),
        out_type=jax.ShapeDtypeStruct((NC, N_NODES, D_EDGE), jnp.float32),
        mesh=mesh,
        scratch_types=[
            pltpu.VMEM((GROUP_ROWS,), jnp.int32),
            pltpu.VMEM((STREAMS_PER_GROUP, ROWS_PER_STREAM), jnp.int32),
            pltpu.VMEM((GROUP_ROWS, D_EDGE), jnp.float32),
            pltpu.VMEM((640, D_EDGE), jnp.float32),
            pltpu.VMEM_SHARED((N_NODES, D_EDGE), jnp.float32),
            pltpu.SemaphoreType.DMA,
        ],
        compiler_params=pltpu.CompilerParams(use_tc_tiling_on_sc=False),
    )(ea_pad, col2d)


TR_BLK = 16000


def _tr_body(ea_t_ref, out_ref):
    t = ea_t_ref[...]
    eye = jnp.eye(D_EDGE, dtype=jnp.float32)
    out_ref[:, pl.ds(0, D_EDGE)] = lax.dot_general(
        t, eye, (((0,), (0,)), ((), ())),
        preferred_element_type=jnp.float32)


@functools.partial(jax.jit, static_argnames=("half",))
def _tc_transpose(ea_t, half):
    nblk = E_HALF // TR_BLK
    blk0 = half * nblk
    return pl.pallas_call(
        _tr_body,
        grid=(nblk,),
        in_specs=[pl.BlockSpec((D_EDGE, TR_BLK), lambda i: (0, i + blk0))],
        out_specs=pl.BlockSpec((TR_BLK, 128), lambda i: (i, 0)),
        out_shape=jax.ShapeDtypeStruct((E_HALF, 128), jnp.float32),
    )(ea_t)


def _col_body(ei_ref, col_ref):
    col_ref[...] = ei_ref[1, :]


@jax.jit
def _tc_col(ei):
    return pl.pallas_call(
        _col_body,
        in_specs=[pl.BlockSpec((2, N_EDGES), lambda: (0, 0))],
        out_specs=pl.BlockSpec((N_EDGES,), lambda: (0,)),
        out_shape=jax.ShapeDtypeStruct((N_EDGES,), jnp.int32),
    )(ei)


def _tc_body(x_ref, pa_ref, pb_ref, batch_ref, u_ref, w1x_ref, w1e_ref,
             w1u_ref, b1_ref, w2_ref, b2_ref, out_ref):
    B = x_ref.shape[0]
    agg = (pa_ref[0] + pa_ref[1]) + (pb_ref[0] + pb_ref[1])
    g = jnp.dot(u_ref[...], w1u_ref[...], preferred_element_type=jnp.float32)
    onehot_t = (batch_ref[0] == lax.broadcasted_iota(jnp.int32, (16, B), 0)
                ).astype(jnp.float32)
    ug = lax.dot_general(onehot_t, g, (((0,), (0,)), ((), ())),
                         preferred_element_type=jnp.float32)
    pre = (jnp.dot(x_ref[...], w1x_ref[...], preferred_element_type=jnp.float32)
           + jnp.dot(agg, w1e_ref[...], preferred_element_type=jnp.float32)
           + ug
           + b1_ref[...])
    h = pre * jax.nn.sigmoid(pre)
    pre2 = jnp.dot(h, w2_ref[...], preferred_element_type=jnp.float32) + b2_ref[...]
    out_ref[...] = pre2 * jax.nn.sigmoid(pre2)


@functools.partial(jax.jit, static_argnames=("block",))
def _tc_mlp(x, parts_a, parts_b, batch2d, u, w1x, w1e, w1u, b1, w2, b2,
            block=1000):
    nblk = N_NODES // block
    k = w1u.shape[1]
    return pl.pallas_call(
        _tc_body,
        grid=(nblk,),
        in_specs=[
            pl.BlockSpec((block, x.shape[1]), lambda i: (i, 0)),
            pl.BlockSpec((NC, block, D_EDGE), lambda i: (0, i, 0)),
            pl.BlockSpec((NC, block, D_EDGE), lambda i: (0, i, 0)),
            pl.BlockSpec((1, 1, block), lambda i: (i, 0, 0)),
            pl.BlockSpec(u.shape, lambda i: (0, 0)),
            pl.BlockSpec(w1x.shape, lambda i: (0, 0)),
            pl.BlockSpec(w1e.shape, lambda i: (0, 0)),
            pl.BlockSpec(w1u.shape, lambda i: (0, 0)),
            pl.BlockSpec(b1.shape, lambda i: (0, 0)),
            pl.BlockSpec(w2.shape, lambda i: (0, 0)),
            pl.BlockSpec(b2.shape, lambda i: (0, 0)),
        ],
        out_specs=pl.BlockSpec((block, k), lambda i: (i, 0)),
        out_shape=jax.ShapeDtypeStruct((N_NODES, k), jnp.float32),
    )(x, parts_a, parts_b, batch2d, u, w1x, w1e, w1u, b1, w2, b2)


def kernel(x, edge_index, edge_attr, u, batch, W1, b1, W2, b2):
    col = _tc_col(edge_index.astype(jnp.int32))
    ea_t = edge_attr.T
    ea_a = _tc_transpose(ea_t, 0)
    parts_a = _sc_scatter_add(ea_a, col, 0)
    ea_b = _tc_transpose(ea_t, 1)
    parts_b = _sc_scatter_add(ea_b, col, E_HALF)

    d_feat = x.shape[1]
    w1x = W1[:d_feat]
    w1e = W1[d_feat:d_feat + D_EDGE]
    w1u = W1[d_feat + D_EDGE:]
    batch2d = batch.astype(jnp.int32).reshape(-1, 1, 1000)
    return _tc_mlp(x, parts_a, parts_b, batch2d, u, w1x, w1e, w1u,
                   b1.reshape(1, -1), W2, b2.reshape(1, -1))
